# Initial kernel scaffold; baseline (speedup 1.0000x reference)
import jax
import jax.numpy as jnp
from jax.experimental import pallas as pl


def kernel(feat, edge_index, W_r, b_r, W_gcn, U_gcn, b_gcn, W_ncn, U_ncn, b_ncn):
    raise NotImplementedError("write your pallas kernel here")


# speedup vs baseline: 3.5952x; 3.5952x over previous
import functools

import jax
import jax.numpy as jnp
from jax import lax
from jax.experimental import pallas as pl
from jax.experimental.pallas import tpu as pltpu
from jax.experimental.pallas import tpu_sc as plsc

N_NODES = 10000
N_EDGES = 160000
D = 256
DH = 128

NPAD = 10240
ZROWS = NPAD // 16
NF = 10008
CHUNK = 128
EPC = 10112
NCHUNK = EPC // CHUNK
EPAD = EPC * 16

BR = 400


def _sc_agg_body(srcp, dstp, fl, fr, z2, z1, onesh,
                 aggl_o, aggr_o, deg_o,
                 src_v, dst_v, rows_v, ones_v, agg_sh, deg_sh, sem):
  cid = lax.axis_index("c")
  sid = lax.axis_index("s")
  r0 = sid * ZROWS

  pltpu.sync_copy(z2, agg_sh.at[pl.ds(r0, ZROWS)])

  @pl.when(cid == 0)
  def _():
    pltpu.sync_copy(z1, deg_sh.at[pl.ds(r0, ZROWS)])

  pltpu.sync_copy(onesh, ones_v)
  plsc.subcore_barrier()

  base = sid * EPC

  def step(i, carry):
    e0 = pl.multiple_of(base + i * CHUNK, CHUNK)
    pltpu.sync_copy(srcp.at[pl.ds(e0, CHUNK)], src_v)
    pltpu.sync_copy(dstp.at[pl.ds(e0, CHUNK)], dst_v)

    @pl.when(cid == 0)
    def _():
      pltpu.async_copy(fl.at[src_v], rows_v, sem).wait()

    @pl.when(cid == 1)
    def _():
      pltpu.async_copy(fr.at[src_v], rows_v, sem).wait()

    pltpu.sync_copy(rows_v, agg_sh.at[dst_v], add=True)

    @pl.when(cid == 0)
    def _():
      pltpu.sync_copy(ones_v, deg_sh.at[dst_v], add=True)

    return carry

  lax.fori_loop(0, NCHUNK, step, 0)
  plsc.subcore_barrier()

  @pl.when(cid == 0)
  def _():
    pltpu.sync_copy(agg_sh.at[pl.ds(r0, ZROWS)], aggl_o.at[pl.ds(r0, ZROWS)])
    pltpu.sync_copy(deg_sh.at[pl.ds(r0, ZROWS)], deg_o.at[pl.ds(r0, ZROWS)])

  @pl.when(cid == 1)
  def _():
    pltpu.sync_copy(agg_sh.at[pl.ds(r0, ZROWS)], aggr_o.at[pl.ds(r0, ZROWS)])


@functools.lru_cache(maxsize=None)
def _build_sc_agg():
  return pl.kernel(
      _sc_agg_body,
      out_type=(
          jax.ShapeDtypeStruct((NPAD, DH), jnp.float32),
          jax.ShapeDtypeStruct((NPAD, DH), jnp.float32),
          jax.ShapeDtypeStruct((NPAD,), jnp.float32),
      ),
      mesh=plsc.VectorSubcoreMesh(core_axis_name="c", subcore_axis_name="s"),
      scratch_types=(
          pltpu.VMEM((CHUNK,), jnp.int32),
          pltpu.VMEM((CHUNK,), jnp.int32),
          pltpu.VMEM((CHUNK, DH), jnp.float32),
          pltpu.VMEM((CHUNK,), jnp.float32),
          pltpu.VMEM_SHARED((NPAD, DH), jnp.float32),
          pltpu.VMEM_SHARED((NPAD,), jnp.float32),
          pltpu.SemaphoreType.DMA,
      ),
  )


def _tc_dense_body(feat_ref, aggl_ref, aggr_ref, deg_ref,
                   wr_ref, br_ref, wg_ref, ug_ref, bg_ref,
                   wn_ref, un_ref, bn_ref, out_ref):
  f = feat_ref[...]
  rdeg = 1.0 / jnp.maximum(deg_ref[...], 1.0)
  al = aggl_ref[...] * rdeg
  ar = aggr_ref[...] * rdeg

  dot = functools.partial(jnp.dot, preferred_element_type=jnp.float32,
                          precision=lax.Precision.HIGHEST)
  gcn = (dot(al, wg_ref[0:DH, :]) + dot(ar, wg_ref[DH:D, :])
         + dot(f, ug_ref[...]) + bg_ref[...])
  ncn = (jnp.maximum(dot(al, wn_ref[0:DH, :]) + dot(ar, wn_ref[DH:D, :]), 0.0)
         + dot(f, un_ref[...]) + bn_ref[...])

  lg = dot(f, wr_ref[...]) + br_ref[...]
  dlt = lg[:, 1:2] - lg[:, 0:1]
  p1 = 1.0 / (1.0 + jnp.exp(-dlt))
  p0 = 1.0 - p1
  out_ref[...] = p0 * gcn + p1 * ncn


_tc_dense = pl.pallas_call(
    _tc_dense_body,
    grid=(N_NODES // BR,),
    in_specs=[
        pl.BlockSpec((BR, D), lambda i: (i, 0)),
        pl.BlockSpec((BR, DH), lambda i: (i, 0)),
        pl.BlockSpec((BR, DH), lambda i: (i, 0)),
        pl.BlockSpec((BR, 1), lambda i: (i, 0)),
        pl.BlockSpec((D, 2), lambda i: (0, 0)),
        pl.BlockSpec((1, 2), lambda i: (0, 0)),
        pl.BlockSpec((D, D), lambda i: (0, 0)),
        pl.BlockSpec((D, D), lambda i: (0, 0)),
        pl.BlockSpec((1, D), lambda i: (0, 0)),
        pl.BlockSpec((D, D), lambda i: (0, 0)),
        pl.BlockSpec((D, D), lambda i: (0, 0)),
        pl.BlockSpec((1, D), lambda i: (0, 0)),
    ],
    out_specs=pl.BlockSpec((BR, D), lambda i: (i, 0)),
    out_shape=jax.ShapeDtypeStruct((N_NODES, D), jnp.float32),
)


def kernel(feat, edge_index, W_r, b_r, W_gcn, U_gcn, b_gcn, W_ncn, U_ncn, b_ncn):
  src = edge_index[0]
  dst = edge_index[1]

  pad = EPAD - N_EDGES
  srcp = jnp.concatenate([src, jnp.full((pad,), N_NODES, jnp.int32)])
  dstp = jnp.concatenate([dst, jnp.full((pad,), N_NODES, jnp.int32)])

  zrow = jnp.zeros((NF - N_NODES, DH), jnp.float32)
  fl = jnp.concatenate([feat[:, :DH], zrow])
  fr = jnp.concatenate([feat[:, DH:], zrow])

  z2 = jnp.zeros((ZROWS, DH), jnp.float32)
  z1 = jnp.zeros((ZROWS,), jnp.float32)
  onesh = jnp.ones((CHUNK,), jnp.float32)

  aggl, aggr, deg = _build_sc_agg()(srcp, dstp, fl, fr, z2, z1, onesh)

  out = _tc_dense(feat, aggl, aggr, deg.reshape(NPAD, 1),
                  W_r, b_r.reshape(1, 2), W_gcn, U_gcn, b_gcn.reshape(1, D),
                  W_ncn, U_ncn, b_ncn.reshape(1, D))
  return out


# speedup vs baseline: 4.1767x; 1.1617x over previous
import functools

import jax
import jax.numpy as jnp
from jax import lax
from jax.experimental import pallas as pl
from jax.experimental.pallas import tpu as pltpu
from jax.experimental.pallas import tpu_sc as plsc

N_NODES = 10000
N_EDGES = 160000
D = 256
DH = 128

NPAD = 10240
ZROWS = NPAD // 16
NF = 10008
CHUNK = 128
NCHUNK = 80
EPC = NCHUNK * CHUNK
EPAD = EPC * 16
NBUF = 4
NOUTER = NCHUNK // NBUF

BR = 400


def _sc_agg_body(srcp, dstp, fl, fr, z2, z1, onesh,
                 aggl_o, aggr_o, deg_o,
                 sv0, sv1, sv2, sv3, dv0, dv1, dv2, dv3,
                 rows0, rows1, ones_v,
                 agg_sh, deg_sh, is0, is1, is2, is3, gs0, gs1, dsem):
  cid = lax.axis_index("c")
  sid = lax.axis_index("s")
  r0 = sid * ZROWS
  srcv = (sv0, sv1, sv2, sv3)
  dstv = (dv0, dv1, dv2, dv3)
  rows = (rows0, rows1)
  isems = (is0, is1, is2, is3)
  gsems = (gs0, gs1)

  pltpu.sync_copy(z2, agg_sh.at[pl.ds(r0, ZROWS)])

  @pl.when(cid == 1)
  def _():
    pltpu.sync_copy(z1, deg_sh.at[pl.ds(r0, ZROWS)])

  pltpu.sync_copy(onesh, ones_v)

  def idx_start(i, b):
    pltpu.async_copy(srcp.at[sid, i], srcv[b], isems[b])
    pltpu.async_copy(dstp.at[sid, i], dstv[b], isems[b])

  def idx_wait(b):
    pltpu.make_async_copy(srcp.at[0, 0], srcv[b], isems[b]).wait()
    pltpu.make_async_copy(dstp.at[0, 0], dstv[b], isems[b]).wait()

  def gather_start(i, b, ib):
    @pl.when(cid == 0)
    def _():
      pltpu.async_copy(fl.at[srcv[ib]], rows[b], gsems[b])

    @pl.when(cid == 1)
    def _():
      pltpu.async_copy(fr.at[srcv[ib]], rows[b], gsems[b])

  def gather_wait(b, ib):
    pltpu.make_async_copy(fl.at[srcv[ib]], rows[b], gsems[b]).wait()

  def deg_start(ib):
    @pl.when(cid == 1)
    def _():
      pltpu.async_copy(ones_v, deg_sh.at[dstv[ib]], dsem, add=True)

  def deg_wait(ib):
    @pl.when(cid == 1)
    def _():
      pltpu.make_async_copy(ones_v, deg_sh.at[dstv[ib]], dsem).wait()

  idx_start(0, 0)
  idx_start(1, 1)
  plsc.subcore_barrier()

  def outer(g, carry):
    for b in range(NBUF):
      i = g * NBUF + b
      rb = b % 2
      pb = (b + 3) % 4

      def when_pos(fn):
        if b == 0:
          pl.when(g > 0)(fn)
        else:
          fn()

      def when_ge2(fn):
        if b < 2:
          pl.when(g > 0)(fn)
        else:
          fn()

      idx_wait(b)
      gather_start(i, rb, b)
      when_pos(lambda: gather_wait(1 - rb, pb))
      when_ge2(lambda: deg_wait((b + 2) % 4))
      when_pos(lambda: pltpu.sync_copy(rows[1 - rb],
                                       agg_sh.at[dstv[pb]], add=True))
      when_pos(lambda: deg_start(pb))
      if b < 2:
        idx_start(i + 2, (b + 2) % 4)
      else:
        @pl.when(g < NOUTER - 1)
        def _():
          idx_start(i + 2, (b + 2) % 4)
    return carry

  lax.fori_loop(0, NOUTER, outer, 0)

  lb = (NCHUNK - 1) % 4
  gather_wait((NCHUNK - 1) % 2, lb)
  deg_wait((lb + 3) % 4)
  pltpu.sync_copy(rows[(NCHUNK - 1) % 2], agg_sh.at[dstv[lb]], add=True)
  deg_start(lb)
  deg_wait(lb)

  plsc.subcore_barrier()

  @pl.when(cid == 0)
  def _():
    pltpu.sync_copy(agg_sh.at[pl.ds(r0, ZROWS)], aggl_o.at[pl.ds(r0, ZROWS)])

  @pl.when(cid == 1)
  def _():
    pltpu.sync_copy(agg_sh.at[pl.ds(r0, ZROWS)], aggr_o.at[pl.ds(r0, ZROWS)])
    pltpu.sync_copy(deg_sh.at[pl.ds(r0, ZROWS)], deg_o.at[pl.ds(r0, ZROWS)])


@functools.lru_cache(maxsize=None)
def _build_sc_agg():
  return pl.kernel(
      _sc_agg_body,
      out_type=(
          jax.ShapeDtypeStruct((NPAD, DH), jnp.float32),
          jax.ShapeDtypeStruct((NPAD, DH), jnp.float32),
          jax.ShapeDtypeStruct((NPAD,), jnp.float32),
      ),
      mesh=plsc.VectorSubcoreMesh(core_axis_name="c", subcore_axis_name="s"),
      scratch_types=(
          pltpu.VMEM((CHUNK,), jnp.int32),
          pltpu.VMEM((CHUNK,), jnp.int32),
          pltpu.VMEM((CHUNK,), jnp.int32),
          pltpu.VMEM((CHUNK,), jnp.int32),
          pltpu.VMEM((CHUNK,), jnp.int32),
          pltpu.VMEM((CHUNK,), jnp.int32),
          pltpu.VMEM((CHUNK,), jnp.int32),
          pltpu.VMEM((CHUNK,), jnp.int32),
          pltpu.VMEM((CHUNK, DH), jnp.float32),
          pltpu.VMEM((CHUNK, DH), jnp.float32),
          pltpu.VMEM((CHUNK,), jnp.float32),
          pltpu.VMEM_SHARED((NPAD, DH), jnp.float32),
          pltpu.VMEM_SHARED((NPAD,), jnp.float32),
          pltpu.SemaphoreType.DMA,
          pltpu.SemaphoreType.DMA,
          pltpu.SemaphoreType.DMA,
          pltpu.SemaphoreType.DMA,
          pltpu.SemaphoreType.DMA,
          pltpu.SemaphoreType.DMA,
          pltpu.SemaphoreType.DMA,
      ),
  )


def _tc_dense_body(feat_ref, aggl_ref, aggr_ref, deg_ref,
                   wr_ref, br_ref, wg_ref, ug_ref, bg_ref,
                   wn_ref, un_ref, bn_ref, out_ref):
  f = feat_ref[...]
  rdeg = 1.0 / jnp.maximum(deg_ref[...], 1.0)
  al = aggl_ref[...] * rdeg
  ar = aggr_ref[...] * rdeg

  dot = functools.partial(jnp.dot, preferred_element_type=jnp.float32,
                          precision=lax.Precision.HIGHEST)
  gcn = (dot(al, wg_ref[0:DH, :]) + dot(ar, wg_ref[DH:D, :])
         + dot(f, ug_ref[...]) + bg_ref[...])
  ncn = (jnp.maximum(dot(al, wn_ref[0:DH, :]) + dot(ar, wn_ref[DH:D, :]), 0.0)
         + dot(f, un_ref[...]) + bn_ref[...])

  lg = dot(f, wr_ref[...]) + br_ref[...]
  dlt = lg[:, 1:2] - lg[:, 0:1]
  p1 = 1.0 / (1.0 + jnp.exp(-dlt))
  p0 = 1.0 - p1
  out_ref[...] = p0 * gcn + p1 * ncn


_tc_dense = pl.pallas_call(
    _tc_dense_body,
    grid=(N_NODES // BR,),
    in_specs=[
        pl.BlockSpec((BR, D), lambda i: (i, 0)),
        pl.BlockSpec((BR, DH), lambda i: (i, 0)),
        pl.BlockSpec((BR, DH), lambda i: (i, 0)),
        pl.BlockSpec((BR, 1), lambda i: (i, 0)),
        pl.BlockSpec((D, 2), lambda i: (0, 0)),
        pl.BlockSpec((1, 2), lambda i: (0, 0)),
        pl.BlockSpec((D, D), lambda i: (0, 0)),
        pl.BlockSpec((D, D), lambda i: (0, 0)),
        pl.BlockSpec((1, D), lambda i: (0, 0)),
        pl.BlockSpec((D, D), lambda i: (0, 0)),
        pl.BlockSpec((D, D), lambda i: (0, 0)),
        pl.BlockSpec((1, D), lambda i: (0, 0)),
    ],
    out_specs=pl.BlockSpec((BR, D), lambda i: (i, 0)),
    out_shape=jax.ShapeDtypeStruct((N_NODES, D), jnp.float32),
)


def kernel(feat, edge_index, W_r, b_r, W_gcn, U_gcn, b_gcn, W_ncn, U_ncn, b_ncn):
  src = edge_index[0]
  dst = edge_index[1]

  pad = EPAD - N_EDGES
  srcp = jnp.concatenate(
      [src, jnp.full((pad,), N_NODES, jnp.int32)]).reshape(16, NCHUNK, CHUNK)
  dstp = jnp.concatenate(
      [dst, jnp.full((pad,), N_NODES, jnp.int32)]).reshape(16, NCHUNK, CHUNK)

  zrow = jnp.zeros((NF - N_NODES, DH), jnp.float32)
  fl = jnp.concatenate([feat[:, :DH], zrow])
  fr = jnp.concatenate([feat[:, DH:], zrow])

  z2 = jnp.zeros((ZROWS, DH), jnp.float32)
  z1 = jnp.zeros((ZROWS,), jnp.float32)
  onesh = jnp.ones((CHUNK,), jnp.float32)

  aggl, aggr, deg = _build_sc_agg()(srcp, dstp, fl, fr, z2, z1, onesh)

  out = _tc_dense(feat, aggl, aggr, deg.reshape(NPAD, 1),
                  W_r, b_r.reshape(1, 2), W_gcn, U_gcn, b_gcn.reshape(1, D),
                  W_ncn, U_ncn, b_ncn.reshape(1, D))
  return out


# speedup vs baseline: 4.2470x; 1.0168x over previous
import functools

import jax
import jax.numpy as jnp
from jax import lax
from jax.experimental import pallas as pl
from jax.experimental.pallas import tpu as pltpu
from jax.experimental.pallas import tpu_sc as plsc

N_NODES = 10000
N_EDGES = 160000
D = 256
DH = 128

NPAD = 10240
ZROWS = NPAD // 16
NF = 10008
CHUNK = 128
NCHUNK = 80
EPC = NCHUNK * CHUNK
EPAD = EPC * 16
NBUF = 4
NOUTER = NCHUNK // NBUF

BR = 400


def _sc_agg_body(srcp, dstp, fl, fr, z2, z1, onesh,
                 aggl_o, aggr_o, deg_o,
                 sv0, sv1, sv2, sv3, dv0, dv1, dv2, dv3,
                 rows0, rows1, ones_v,
                 agg_sh, deg_sh, is0, is1, is2, is3, gs0, gs1, dsem):
  cid = lax.axis_index("c")
  sid = lax.axis_index("s")
  r0 = sid * ZROWS
  srcv = (sv0, sv1, sv2, sv3)
  dstv = (dv0, dv1, dv2, dv3)
  rows = (rows0, rows1)
  isems = (is0, is1, is2, is3)
  gsems = (gs0, gs1)

  pltpu.sync_copy(z2, agg_sh.at[pl.ds(r0, ZROWS)])

  @pl.when(cid == 1)
  def _():
    pltpu.sync_copy(z1, deg_sh.at[pl.ds(r0, ZROWS)])

  pltpu.sync_copy(onesh, ones_v)

  def idx_start(i, b):
    pltpu.async_copy(srcp.at[sid, i], srcv[b], isems[b])
    pltpu.async_copy(dstp.at[sid, i], dstv[b], isems[b])

  def idx_wait(b):
    pltpu.make_async_copy(srcp.at[0, 0], srcv[b], isems[b]).wait()
    pltpu.make_async_copy(dstp.at[0, 0], dstv[b], isems[b]).wait()

  def gather_start(i, b, ib):
    @pl.when(cid == 0)
    def _():
      pltpu.async_copy(fl.at[srcv[ib]], rows[b], gsems[b])

    @pl.when(cid == 1)
    def _():
      pltpu.async_copy(fr.at[srcv[ib]], rows[b], gsems[b])

  def gather_wait(b, ib):
    pltpu.make_async_copy(fl.at[srcv[ib]], rows[b], gsems[b]).wait()

  def deg_start(ib):
    @pl.when(cid == 1)
    def _():
      pltpu.async_copy(ones_v, deg_sh.at[dstv[ib]], dsem, add=True)

  def deg_wait(ib):
    @pl.when(cid == 1)
    def _():
      pltpu.make_async_copy(ones_v, deg_sh.at[dstv[ib]], dsem).wait()

  idx_start(0, 0)
  idx_start(1, 1)
  plsc.subcore_barrier()

  def outer(g, carry):
    for b in range(NBUF):
      i = g * NBUF + b
      rb = b % 2
      pb = (b + 3) % 4

      def when_pos(fn):
        if b == 0:
          pl.when(g > 0)(fn)
        else:
          fn()

      def when_ge2(fn):
        if b < 2:
          pl.when(g > 0)(fn)
        else:
          fn()

      idx_wait(b)
      gather_start(i, rb, b)
      when_pos(lambda: gather_wait(1 - rb, pb))
      when_ge2(lambda: deg_wait((b + 2) % 4))
      when_pos(lambda: pltpu.sync_copy(rows[1 - rb],
                                       agg_sh.at[dstv[pb]], add=True))
      when_pos(lambda: deg_start(pb))
      if b < 2:
        idx_start(i + 2, (b + 2) % 4)
      else:
        @pl.when(g < NOUTER - 1)
        def _():
          idx_start(i + 2, (b + 2) % 4)
    return carry

  lax.fori_loop(0, NOUTER, outer, 0)

  lb = (NCHUNK - 1) % 4
  gather_wait((NCHUNK - 1) % 2, lb)
  deg_wait((lb + 3) % 4)
  pltpu.sync_copy(rows[(NCHUNK - 1) % 2], agg_sh.at[dstv[lb]], add=True)
  deg_start(lb)
  deg_wait(lb)

  plsc.subcore_barrier()

  @pl.when(cid == 0)
  def _():
    pltpu.sync_copy(agg_sh.at[pl.ds(r0, ZROWS)], aggl_o.at[pl.ds(r0, ZROWS)])

  @pl.when(cid == 1)
  def _():
    pltpu.sync_copy(agg_sh.at[pl.ds(r0, ZROWS)], aggr_o.at[pl.ds(r0, ZROWS)])
    pltpu.sync_copy(deg_sh.at[pl.ds(r0, ZROWS)], deg_o.at[pl.ds(r0, ZROWS)])


@functools.lru_cache(maxsize=None)
def _build_sc_agg():
  return pl.kernel(
      _sc_agg_body,
      out_type=(
          jax.ShapeDtypeStruct((NPAD, DH), jnp.float32),
          jax.ShapeDtypeStruct((NPAD, DH), jnp.float32),
          jax.ShapeDtypeStruct((NPAD,), jnp.float32),
      ),
      mesh=plsc.VectorSubcoreMesh(core_axis_name="c", subcore_axis_name="s"),
      scratch_types=(
          pltpu.VMEM((CHUNK,), jnp.int32),
          pltpu.VMEM((CHUNK,), jnp.int32),
          pltpu.VMEM((CHUNK,), jnp.int32),
          pltpu.VMEM((CHUNK,), jnp.int32),
          pltpu.VMEM((CHUNK,), jnp.int32),
          pltpu.VMEM((CHUNK,), jnp.int32),
          pltpu.VMEM((CHUNK,), jnp.int32),
          pltpu.VMEM((CHUNK,), jnp.int32),
          pltpu.VMEM((CHUNK, DH), jnp.float32),
          pltpu.VMEM((CHUNK, DH), jnp.float32),
          pltpu.VMEM((CHUNK,), jnp.float32),
          pltpu.VMEM_SHARED((NPAD, DH), jnp.float32),
          pltpu.VMEM_SHARED((NPAD,), jnp.float32),
          pltpu.SemaphoreType.DMA,
          pltpu.SemaphoreType.DMA,
          pltpu.SemaphoreType.DMA,
          pltpu.SemaphoreType.DMA,
          pltpu.SemaphoreType.DMA,
          pltpu.SemaphoreType.DMA,
          pltpu.SemaphoreType.DMA,
      ),
  )


def _tc_base_body(feat_ref, wr_ref, br_ref, ug_ref, bg_ref, un_ref, bn_ref,
                  base_ref, p1_ref):
  f = feat_ref[...]
  dot = functools.partial(jnp.dot, preferred_element_type=jnp.float32,
                          precision=lax.Precision.HIGHEST)
  lg = dot(f, wr_ref[...]) + br_ref[...]
  dlt = lg[:, 1:2] - lg[:, 0:1]
  p1 = 1.0 / (1.0 + jnp.exp(-dlt))
  p0 = 1.0 - p1
  base_ref[...] = (p0 * (dot(f, ug_ref[...]) + bg_ref[...])
                   + p1 * (dot(f, un_ref[...]) + bn_ref[...]))
  p1_ref[...] = p1


_tc_base = pl.pallas_call(
    _tc_base_body,
    grid=(N_NODES // BR,),
    in_specs=[
        pl.BlockSpec((BR, D), lambda i: (i, 0)),
        pl.BlockSpec((D, 2), lambda i: (0, 0)),
        pl.BlockSpec((1, 2), lambda i: (0, 0)),
        pl.BlockSpec((D, D), lambda i: (0, 0)),
        pl.BlockSpec((1, D), lambda i: (0, 0)),
        pl.BlockSpec((D, D), lambda i: (0, 0)),
        pl.BlockSpec((1, D), lambda i: (0, 0)),
    ],
    out_specs=[
        pl.BlockSpec((BR, D), lambda i: (i, 0)),
        pl.BlockSpec((BR, 1), lambda i: (i, 0)),
    ],
    out_shape=[
        jax.ShapeDtypeStruct((N_NODES, D), jnp.float32),
        jax.ShapeDtypeStruct((N_NODES, 1), jnp.float32),
    ],
)


def _tc_comb_body(aggl_ref, aggr_ref, deg_ref, base_ref, p1_ref,
                  wg_ref, wn_ref, out_ref):
  rdeg = 1.0 / jnp.maximum(deg_ref[...], 1.0)
  al = aggl_ref[...] * rdeg
  ar = aggr_ref[...] * rdeg
  p1 = p1_ref[...]
  dot = functools.partial(jnp.dot, preferred_element_type=jnp.float32,
                          precision=lax.Precision.HIGHEST)
  gcn = dot(al, wg_ref[0:DH, :]) + dot(ar, wg_ref[DH:D, :])
  ncn = jnp.maximum(dot(al, wn_ref[0:DH, :]) + dot(ar, wn_ref[DH:D, :]), 0.0)
  out_ref[...] = base_ref[...] + (1.0 - p1) * gcn + p1 * ncn


_tc_comb = pl.pallas_call(
    _tc_comb_body,
    grid=(N_NODES // BR,),
    in_specs=[
        pl.BlockSpec((BR, DH), lambda i: (i, 0)),
        pl.BlockSpec((BR, DH), lambda i: (i, 0)),
        pl.BlockSpec((BR, 1), lambda i: (i, 0)),
        pl.BlockSpec((BR, D), lambda i: (i, 0)),
        pl.BlockSpec((BR, 1), lambda i: (i, 0)),
        pl.BlockSpec((D, D), lambda i: (0, 0)),
        pl.BlockSpec((D, D), lambda i: (0, 0)),
    ],
    out_specs=pl.BlockSpec((BR, D), lambda i: (i, 0)),
    out_shape=jax.ShapeDtypeStruct((N_NODES, D), jnp.float32),
)


def kernel(feat, edge_index, W_r, b_r, W_gcn, U_gcn, b_gcn, W_ncn, U_ncn, b_ncn):
  src = edge_index[0]
  dst = edge_index[1]

  pad = EPAD - N_EDGES
  srcp = jnp.concatenate(
      [src, jnp.full((pad,), N_NODES, jnp.int32)]).reshape(16, NCHUNK, CHUNK)
  dstp = jnp.concatenate(
      [dst, jnp.full((pad,), N_NODES, jnp.int32)]).reshape(16, NCHUNK, CHUNK)

  zrow = jnp.zeros((NF - N_NODES, DH), jnp.float32)
  fl = jnp.concatenate([feat[:, :DH], zrow])
  fr = jnp.concatenate([feat[:, DH:], zrow])

  z2 = jnp.zeros((ZROWS, DH), jnp.float32)
  z1 = jnp.zeros((ZROWS,), jnp.float32)
  onesh = jnp.ones((CHUNK,), jnp.float32)

  aggl, aggr, deg = _build_sc_agg()(srcp, dstp, fl, fr, z2, z1, onesh)

  base, p1 = _tc_base(feat, W_r, b_r.reshape(1, 2), U_gcn, b_gcn.reshape(1, D),
                      U_ncn, b_ncn.reshape(1, D))

  out = _tc_comb(aggl, aggr, deg.reshape(NPAD, 1), base, p1, W_gcn, W_ncn)
  return out


# speedup vs baseline: 6.2208x; 1.4647x over previous
import functools

import jax
import jax.numpy as jnp
from jax import lax
from jax.experimental import pallas as pl
from jax.experimental.pallas import tpu as pltpu
from jax.experimental.pallas import tpu_sc as plsc

N_NODES = 10000
N_EDGES = 160000
D = 256
DH = 128

NPAD = 10240
ZROWS = NPAD // 16
NF = 10008
CHUNK = 112
NCHUNK = 90
EPC = NCHUNK * CHUNK
EPAD = EPC * 16
RB = 3
IR = 6
UNROLL = 6
NOUTER = NCHUNK // UNROLL

BR = 400


def _sc_agg_body(srcp, dstp, fl, fr, z2, z1, onesh,
                 aggl_o, aggr_o, deg0_o, deg1_o,
                 sv0, sv1, sv2, sv3, sv4, sv5,
                 dv0, dv1, dv2, dv3, dv4, dv5,
                 rows0, rows1, rows2, ones_v,
                 agg_sh, deg_sh,
                 is0, is1, is2, is3, is4, is5,
                 gs0, gs1, gs2, ss0, ss1, ss2, dsem):
  cid = lax.axis_index("c")
  sid = lax.axis_index("s")
  r0 = sid * ZROWS
  srcv = (sv0, sv1, sv2, sv3, sv4, sv5)
  dstv = (dv0, dv1, dv2, dv3, dv4, dv5)
  rows = (rows0, rows1, rows2)
  isems = (is0, is1, is2, is3, is4, is5)
  gsems = (gs0, gs1, gs2)
  ssems = (ss0, ss1, ss2)

  pltpu.sync_copy(z2, agg_sh.at[pl.ds(r0, ZROWS)])
  pltpu.sync_copy(z1, deg_sh.at[pl.ds(r0, ZROWS)])
  pltpu.sync_copy(onesh, ones_v)

  def idx_start(i, b):
    pltpu.async_copy(srcp.at[sid, i], srcv[b], isems[b])
    pltpu.async_copy(dstp.at[sid, i], dstv[b], isems[b])

  def idx_wait(b):
    pltpu.make_async_copy(srcp.at[0, 0], srcv[b], isems[b]).wait()
    pltpu.make_async_copy(dstp.at[0, 0], dstv[b], isems[b]).wait()

  def gather_start(i, b, ib):
    @pl.when(cid == 0)
    def _():
      pltpu.async_copy(fl.at[srcv[ib]], rows[b], gsems[b])

    @pl.when(cid == 1)
    def _():
      pltpu.async_copy(fr.at[srcv[ib]], rows[b], gsems[b])

  def gather_wait(b, ib):
    pltpu.make_async_copy(fl.at[srcv[ib]], rows[b], gsems[b]).wait()

  def scatter_start(b, ib):
    pltpu.async_copy(rows[b], agg_sh.at[dstv[ib]], ssems[b], add=True)

  def scatter_wait(b, ib):
    pltpu.make_async_copy(rows[b], agg_sh.at[dstv[ib]], ssems[b]).wait()

  def deg_start(core, ib):
    @pl.when(cid == core)
    def _():
      pltpu.async_copy(ones_v, deg_sh.at[dstv[ib]], dsem, add=True)

  def deg_wait(core, ib):
    @pl.when(cid == core)
    def _():
      pltpu.make_async_copy(ones_v, deg_sh.at[dstv[ib]], dsem).wait()

  idx_start(0, 0)
  idx_start(1, 1)
  plsc.subcore_barrier()

  def outer(g, carry):
    for u in range(UNROLL):
      i = g * UNROLL + u
      b3 = u % RB
      pb3 = (u + RB - 1) % RB
      pb6 = (u + IR - 1) % IR

      def guard(cond_first, fn):
        if cond_first:
          fn()
        else:
          pl.when(g > 0)(fn)

      idx_wait(u)
      guard(u >= 3, lambda: scatter_wait(b3, (u + IR - 3) % IR))
      gather_start(i, b3, u)
      guard(u >= 1, lambda: gather_wait(pb3, pb6))
      guard(u >= 1, lambda: scatter_start(pb3, pb6))
      dcore = (u + 1) % 2
      guard(u >= 3, lambda: deg_wait(dcore, (u + IR - 3) % IR))
      guard(u >= 1, lambda: deg_start(dcore, pb6))
      if u < 4:
        idx_start(i + 2, (u + 2) % IR)
      else:
        @pl.when(g < NOUTER - 1)
        def _():
          idx_start(i + 2, (u + 2) % IR)
    return carry

  lax.fori_loop(0, NOUTER, outer, 0)

  L = NCHUNK - 1
  gather_wait(L % RB, L % IR)
  scatter_wait((L - 2) % RB, (L - 2) % IR)
  scatter_start(L % RB, L % IR)
  deg_wait(0, (L - 1) % IR)
  deg_wait(1, (L - 2) % IR)
  deg_start(1, L % IR)
  deg_wait(1, L % IR)
  scatter_wait((L - 1) % RB, (L - 1) % IR)
  scatter_wait(L % RB, L % IR)

  plsc.subcore_barrier()

  @pl.when(cid == 0)
  def _():
    pltpu.sync_copy(agg_sh.at[pl.ds(r0, ZROWS)], aggl_o.at[pl.ds(r0, ZROWS)])
    pltpu.sync_copy(deg_sh.at[pl.ds(r0, ZROWS)], deg0_o.at[pl.ds(r0, ZROWS)])

  @pl.when(cid == 1)
  def _():
    pltpu.sync_copy(agg_sh.at[pl.ds(r0, ZROWS)], aggr_o.at[pl.ds(r0, ZROWS)])
    pltpu.sync_copy(deg_sh.at[pl.ds(r0, ZROWS)], deg1_o.at[pl.ds(r0, ZROWS)])


@functools.lru_cache(maxsize=None)
def _build_sc_agg():
  return pl.kernel(
      _sc_agg_body,
      out_type=(
          jax.ShapeDtypeStruct((NPAD, DH), jnp.float32),
          jax.ShapeDtypeStruct((NPAD, DH), jnp.float32),
          jax.ShapeDtypeStruct((NPAD,), jnp.float32),
          jax.ShapeDtypeStruct((NPAD,), jnp.float32),
      ),
      mesh=plsc.VectorSubcoreMesh(core_axis_name="c", subcore_axis_name="s"),
      scratch_types=(
          pltpu.VMEM((CHUNK,), jnp.int32),
          pltpu.VMEM((CHUNK,), jnp.int32),
          pltpu.VMEM((CHUNK,), jnp.int32),
          pltpu.VMEM((CHUNK,), jnp.int32),
          pltpu.VMEM((CHUNK,), jnp.int32),
          pltpu.VMEM((CHUNK,), jnp.int32),
          pltpu.VMEM((CHUNK,), jnp.int32),
          pltpu.VMEM((CHUNK,), jnp.int32),
          pltpu.VMEM((CHUNK,), jnp.int32),
          pltpu.VMEM((CHUNK,), jnp.int32),
          pltpu.VMEM((CHUNK,), jnp.int32),
          pltpu.VMEM((CHUNK,), jnp.int32),
          pltpu.VMEM((CHUNK, DH), jnp.float32),
          pltpu.VMEM((CHUNK, DH), jnp.float32),
          pltpu.VMEM((CHUNK, DH), jnp.float32),
          pltpu.VMEM((CHUNK,), jnp.float32),
          pltpu.VMEM_SHARED((NPAD, DH), jnp.float32),
          pltpu.VMEM_SHARED((NPAD,), jnp.float32),
          pltpu.SemaphoreType.DMA,
          pltpu.SemaphoreType.DMA,
          pltpu.SemaphoreType.DMA,
          pltpu.SemaphoreType.DMA,
          pltpu.SemaphoreType.DMA,
          pltpu.SemaphoreType.DMA,
          pltpu.SemaphoreType.DMA,
          pltpu.SemaphoreType.DMA,
          pltpu.SemaphoreType.DMA,
          pltpu.SemaphoreType.DMA,
          pltpu.SemaphoreType.DMA,
          pltpu.SemaphoreType.DMA,
          pltpu.SemaphoreType.DMA,
      ),
  )


def _tc_base_body(feat_ref, wr_ref, br_ref, ug_ref, bg_ref, un_ref, bn_ref,
                  base_ref, p1_ref):
  f = feat_ref[...]
  dot = functools.partial(jnp.dot, preferred_element_type=jnp.float32,
                          precision=lax.Precision.HIGHEST)
  lg = dot(f, wr_ref[...]) + br_ref[...]
  dlt = lg[:, 1:2] - lg[:, 0:1]
  p1 = 1.0 / (1.0 + jnp.exp(-dlt))
  p0 = 1.0 - p1
  base_ref[...] = (p0 * (dot(f, ug_ref[...]) + bg_ref[...])
                   + p1 * (dot(f, un_ref[...]) + bn_ref[...]))
  p1_ref[...] = p1


_tc_base = pl.pallas_call(
    _tc_base_body,
    grid=(N_NODES // BR,),
    in_specs=[
        pl.BlockSpec((BR, D), lambda i: (i, 0)),
        pl.BlockSpec((D, 2), lambda i: (0, 0)),
        pl.BlockSpec((1, 2), lambda i: (0, 0)),
        pl.BlockSpec((D, D), lambda i: (0, 0)),
        pl.BlockSpec((1, D), lambda i: (0, 0)),
        pl.BlockSpec((D, D), lambda i: (0, 0)),
        pl.BlockSpec((1, D), lambda i: (0, 0)),
    ],
    out_specs=[
        pl.BlockSpec((BR, D), lambda i: (i, 0)),
        pl.BlockSpec((BR, 1), lambda i: (i, 0)),
    ],
    out_shape=[
        jax.ShapeDtypeStruct((N_NODES, D), jnp.float32),
        jax.ShapeDtypeStruct((N_NODES, 1), jnp.float32),
    ],
)


def _tc_comb_body(aggl_ref, aggr_ref, deg0_ref, deg1_ref, base_ref, p1_ref,
                  wg_ref, wn_ref, out_ref):
  rdeg = 1.0 / jnp.maximum(deg0_ref[...] + deg1_ref[...], 1.0)
  al = aggl_ref[...] * rdeg
  ar = aggr_ref[...] * rdeg
  p1 = p1_ref[...]
  dot = functools.partial(jnp.dot, preferred_element_type=jnp.float32,
                          precision=lax.Precision.HIGHEST)
  gcn = dot(al, wg_ref[0:DH, :]) + dot(ar, wg_ref[DH:D, :])
  ncn = jnp.maximum(dot(al, wn_ref[0:DH, :]) + dot(ar, wn_ref[DH:D, :]), 0.0)
  out_ref[...] = base_ref[...] + (1.0 - p1) * gcn + p1 * ncn


_tc_comb = pl.pallas_call(
    _tc_comb_body,
    grid=(N_NODES // BR,),
    in_specs=[
        pl.BlockSpec((BR, DH), lambda i: (i, 0)),
        pl.BlockSpec((BR, DH), lambda i: (i, 0)),
        pl.BlockSpec((BR, 1), lambda i: (i, 0)),
        pl.BlockSpec((BR, 1), lambda i: (i, 0)),
        pl.BlockSpec((BR, D), lambda i: (i, 0)),
        pl.BlockSpec((BR, 1), lambda i: (i, 0)),
        pl.BlockSpec((D, D), lambda i: (0, 0)),
        pl.BlockSpec((D, D), lambda i: (0, 0)),
    ],
    out_specs=pl.BlockSpec((BR, D), lambda i: (i, 0)),
    out_shape=jax.ShapeDtypeStruct((N_NODES, D), jnp.float32),
)


def kernel(feat, edge_index, W_r, b_r, W_gcn, U_gcn, b_gcn, W_ncn, U_ncn, b_ncn):
  src = edge_index[0]
  dst = edge_index[1]

  pad = EPAD - N_EDGES
  srcp = jnp.concatenate(
      [src, jnp.full((pad,), N_NODES, jnp.int32)]).reshape(16, NCHUNK, CHUNK)
  dstp = jnp.concatenate(
      [dst, jnp.full((pad,), N_NODES, jnp.int32)]).reshape(16, NCHUNK, CHUNK)

  zrow = jnp.zeros((NF - N_NODES, DH), jnp.float32)
  fl = jnp.concatenate([feat[:, :DH], zrow])
  fr = jnp.concatenate([feat[:, DH:], zrow])

  z2 = jnp.zeros((ZROWS, DH), jnp.float32)
  z1 = jnp.zeros((ZROWS,), jnp.float32)
  onesh = jnp.ones((CHUNK,), jnp.float32)

  aggl, aggr, deg0, deg1 = _build_sc_agg()(srcp, dstp, fl, fr, z2, z1, onesh)

  base, p1 = _tc_base(feat, W_r, b_r.reshape(1, 2), U_gcn, b_gcn.reshape(1, D),
                      U_ncn, b_ncn.reshape(1, D))

  out = _tc_comb(aggl, aggr, deg0.reshape(NPAD, 1), deg1.reshape(NPAD, 1),
                 base, p1, W_gcn, W_ncn)
  return out


# speedup vs baseline: 6.3409x; 1.0193x over previous
import functools

import jax
import jax.numpy as jnp
from jax import lax
from jax.experimental import pallas as pl
from jax.experimental.pallas import tpu as pltpu
from jax.experimental.pallas import tpu_sc as plsc

N_NODES = 10000
N_EDGES = 160000
D = 256
DH = 128

NPAD = 10240
ZROWS = NPAD // 16
CHUNK = 112
NCHUNK = 90
EPC = NCHUNK * CHUNK
EPAD = EPC * 16
RB = 3
IR = 6
UNROLL = 6
NOUTER = NCHUNK // UNROLL

BR = 400


def _sc_agg_body(srcp, dstp, ff, z2, z1, onesh,
                 aggl_o, aggr_o, deg0_o, deg1_o,
                 sv0, sv1, sv2, sv3, sv4, sv5,
                 dv0, dv1, dv2, dv3, dv4, dv5,
                 rows0, rows1, rows2, ones_v,
                 agg_sh, deg_sh,
                 is0, is1, is2, is3, is4, is5,
                 gs0, gs1, gs2, ss0, ss1, ss2, dsem):
  cid = lax.axis_index("c")
  sid = lax.axis_index("s")
  r0 = sid * ZROWS
  srcv = (sv0, sv1, sv2, sv3, sv4, sv5)
  dstv = (dv0, dv1, dv2, dv3, dv4, dv5)
  rows = (rows0, rows1, rows2)
  isems = (is0, is1, is2, is3, is4, is5)
  gsems = (gs0, gs1, gs2)
  ssems = (ss0, ss1, ss2)

  pltpu.sync_copy(z2, agg_sh.at[pl.ds(r0, ZROWS)])
  pltpu.sync_copy(z1, deg_sh.at[pl.ds(r0, ZROWS)])
  pltpu.sync_copy(onesh, ones_v)

  def idx_start(i, b):
    pltpu.async_copy(srcp.at[sid, i], srcv[b], isems[b])
    pltpu.async_copy(dstp.at[sid, i], dstv[b], isems[b])

  def idx_wait(b):
    pltpu.make_async_copy(srcp.at[0, 0], srcv[b], isems[b]).wait()
    pltpu.make_async_copy(dstp.at[0, 0], dstv[b], isems[b]).wait()
    for j in range(CHUNK // 16):
      sl = pl.ds(j * 16, 16)
      srcv[b][sl] = srcv[b][sl] * 2 + cid

  def gather_start(i, b, ib):
    pltpu.async_copy(ff.at[srcv[ib]], rows[b], gsems[b])

  def gather_wait(b, ib):
    pltpu.make_async_copy(ff.at[srcv[ib]], rows[b], gsems[b]).wait()

  def scatter_start(b, ib):
    pltpu.async_copy(rows[b], agg_sh.at[dstv[ib]], ssems[b], add=True)

  def scatter_wait(b, ib):
    pltpu.make_async_copy(rows[b], agg_sh.at[dstv[ib]], ssems[b]).wait()

  def deg_start(core, ib):
    @pl.when(cid == core)
    def _():
      pltpu.async_copy(ones_v, deg_sh.at[dstv[ib]], dsem, add=True)

  def deg_wait(core, ib):
    @pl.when(cid == core)
    def _():
      pltpu.make_async_copy(ones_v, deg_sh.at[dstv[ib]], dsem).wait()

  idx_start(0, 0)
  idx_start(1, 1)
  plsc.subcore_barrier()

  def outer(g, carry):
    for u in range(UNROLL):
      i = g * UNROLL + u
      b3 = u % RB
      pb3 = (u + RB - 1) % RB
      pb6 = (u + IR - 1) % IR

      def guard(cond_first, fn):
        if cond_first:
          fn()
        else:
          pl.when(g > 0)(fn)

      idx_wait(u)
      guard(u >= 3, lambda: scatter_wait(b3, (u + IR - 3) % IR))
      gather_start(i, b3, u)
      guard(u >= 1, lambda: gather_wait(pb3, pb6))
      guard(u >= 1, lambda: scatter_start(pb3, pb6))
      dcore = (u + 1) % 2
      guard(u >= 3, lambda: deg_wait(dcore, (u + IR - 3) % IR))
      guard(u >= 1, lambda: deg_start(dcore, pb6))
      if u < 4:
        idx_start(i + 2, (u + 2) % IR)
      else:
        @pl.when(g < NOUTER - 1)
        def _():
          idx_start(i + 2, (u + 2) % IR)
    return carry

  lax.fori_loop(0, NOUTER, outer, 0)

  L = NCHUNK - 1
  gather_wait(L % RB, L % IR)
  scatter_wait((L - 2) % RB, (L - 2) % IR)
  scatter_start(L % RB, L % IR)
  deg_wait(0, (L - 1) % IR)
  deg_wait(1, (L - 2) % IR)
  deg_start(1, L % IR)
  deg_wait(1, L % IR)
  scatter_wait((L - 1) % RB, (L - 1) % IR)
  scatter_wait(L % RB, L % IR)

  plsc.subcore_barrier()

  @pl.when(cid == 0)
  def _():
    pltpu.sync_copy(agg_sh.at[pl.ds(r0, ZROWS)], aggl_o.at[pl.ds(r0, ZROWS)])
    pltpu.sync_copy(deg_sh.at[pl.ds(r0, ZROWS)], deg0_o.at[pl.ds(r0, ZROWS)])

  @pl.when(cid == 1)
  def _():
    pltpu.sync_copy(agg_sh.at[pl.ds(r0, ZROWS)], aggr_o.at[pl.ds(r0, ZROWS)])
    pltpu.sync_copy(deg_sh.at[pl.ds(r0, ZROWS)], deg1_o.at[pl.ds(r0, ZROWS)])


@functools.lru_cache(maxsize=None)
def _build_sc_agg():
  return pl.kernel(
      _sc_agg_body,
      out_type=(
          jax.ShapeDtypeStruct((NPAD, DH), jnp.float32),
          jax.ShapeDtypeStruct((NPAD, DH), jnp.float32),
          jax.ShapeDtypeStruct((NPAD,), jnp.float32),
          jax.ShapeDtypeStruct((NPAD,), jnp.float32),
      ),
      mesh=plsc.VectorSubcoreMesh(core_axis_name="c", subcore_axis_name="s"),
      scratch_types=(
          pltpu.VMEM((CHUNK,), jnp.int32),
          pltpu.VMEM((CHUNK,), jnp.int32),
          pltpu.VMEM((CHUNK,), jnp.int32),
          pltpu.VMEM((CHUNK,), jnp.int32),
          pltpu.VMEM((CHUNK,), jnp.int32),
          pltpu.VMEM((CHUNK,), jnp.int32),
          pltpu.VMEM((CHUNK,), jnp.int32),
          pltpu.VMEM((CHUNK,), jnp.int32),
          pltpu.VMEM((CHUNK,), jnp.int32),
          pltpu.VMEM((CHUNK,), jnp.int32),
          pltpu.VMEM((CHUNK,), jnp.int32),
          pltpu.VMEM((CHUNK,), jnp.int32),
          pltpu.VMEM((CHUNK, DH), jnp.float32),
          pltpu.VMEM((CHUNK, DH), jnp.float32),
          pltpu.VMEM((CHUNK, DH), jnp.float32),
          pltpu.VMEM((CHUNK,), jnp.float32),
          pltpu.VMEM_SHARED((NPAD, DH), jnp.float32),
          pltpu.VMEM_SHARED((NPAD,), jnp.float32),
          pltpu.SemaphoreType.DMA,
          pltpu.SemaphoreType.DMA,
          pltpu.SemaphoreType.DMA,
          pltpu.SemaphoreType.DMA,
          pltpu.SemaphoreType.DMA,
          pltpu.SemaphoreType.DMA,
          pltpu.SemaphoreType.DMA,
          pltpu.SemaphoreType.DMA,
          pltpu.SemaphoreType.DMA,
          pltpu.SemaphoreType.DMA,
          pltpu.SemaphoreType.DMA,
          pltpu.SemaphoreType.DMA,
          pltpu.SemaphoreType.DMA,
      ),
  )


def _tc_base_body(feat_ref, wr_ref, br_ref, ug_ref, bg_ref, un_ref, bn_ref,
                  base_ref, p1_ref):
  f = feat_ref[...]
  dot = functools.partial(jnp.dot, preferred_element_type=jnp.float32,
                          precision=lax.Precision.DEFAULT)
  lg = dot(f, wr_ref[...]) + br_ref[...]
  dlt = lg[:, 1:2] - lg[:, 0:1]
  p1 = 1.0 / (1.0 + jnp.exp(-dlt))
  p0 = 1.0 - p1
  base_ref[...] = (p0 * (dot(f, ug_ref[...]) + bg_ref[...])
                   + p1 * (dot(f, un_ref[...]) + bn_ref[...]))
  p1_ref[...] = p1


_tc_base = pl.pallas_call(
    _tc_base_body,
    grid=(N_NODES // BR,),
    in_specs=[
        pl.BlockSpec((BR, D), lambda i: (i, 0)),
        pl.BlockSpec((D, 2), lambda i: (0, 0)),
        pl.BlockSpec((1, 2), lambda i: (0, 0)),
        pl.BlockSpec((D, D), lambda i: (0, 0)),
        pl.BlockSpec((1, D), lambda i: (0, 0)),
        pl.BlockSpec((D, D), lambda i: (0, 0)),
        pl.BlockSpec((1, D), lambda i: (0, 0)),
    ],
    out_specs=[
        pl.BlockSpec((BR, D), lambda i: (i, 0)),
        pl.BlockSpec((BR, 1), lambda i: (i, 0)),
    ],
    out_shape=[
        jax.ShapeDtypeStruct((N_NODES, D), jnp.float32),
        jax.ShapeDtypeStruct((N_NODES, 1), jnp.float32),
    ],
)


def _tc_comb_body(aggl_ref, aggr_ref, deg0_ref, deg1_ref, base_ref, p1_ref,
                  wg_ref, wn_ref, out_ref):
  rdeg = 1.0 / jnp.maximum(deg0_ref[...] + deg1_ref[...], 1.0)
  al = aggl_ref[...] * rdeg
  ar = aggr_ref[...] * rdeg
  p1 = p1_ref[...]
  dot = functools.partial(jnp.dot, preferred_element_type=jnp.float32,
                          precision=lax.Precision.DEFAULT)
  gcn = dot(al, wg_ref[0:DH, :]) + dot(ar, wg_ref[DH:D, :])
  ncn = jnp.maximum(dot(al, wn_ref[0:DH, :]) + dot(ar, wn_ref[DH:D, :]), 0.0)
  out_ref[...] = base_ref[...] + (1.0 - p1) * gcn + p1 * ncn


_tc_comb = pl.pallas_call(
    _tc_comb_body,
    grid=(N_NODES // BR,),
    in_specs=[
        pl.BlockSpec((BR, DH), lambda i: (i, 0)),
        pl.BlockSpec((BR, DH), lambda i: (i, 0)),
        pl.BlockSpec((BR, 1), lambda i: (i, 0)),
        pl.BlockSpec((BR, 1), lambda i: (i, 0)),
        pl.BlockSpec((BR, D), lambda i: (i, 0)),
        pl.BlockSpec((BR, 1), lambda i: (i, 0)),
        pl.BlockSpec((D, D), lambda i: (0, 0)),
        pl.BlockSpec((D, D), lambda i: (0, 0)),
    ],
    out_specs=pl.BlockSpec((BR, D), lambda i: (i, 0)),
    out_shape=jax.ShapeDtypeStruct((N_NODES, D), jnp.float32),
)


def kernel(feat, edge_index, W_r, b_r, W_gcn, U_gcn, b_gcn, W_ncn, U_ncn, b_ncn):
  src = edge_index[0]
  dst = edge_index[1]

  pad = EPAD - N_EDGES
  srcp = jnp.concatenate(
      [src, jnp.zeros((pad,), jnp.int32)]).reshape(16, NCHUNK, CHUNK)
  dstp = jnp.concatenate(
      [dst, jnp.full((pad,), N_NODES, jnp.int32)]).reshape(16, NCHUNK, CHUNK)

  ff = feat.reshape(2 * N_NODES, DH)

  z2 = jnp.zeros((ZROWS, DH), jnp.float32)
  z1 = jnp.zeros((ZROWS,), jnp.float32)
  onesh = jnp.ones((CHUNK,), jnp.float32)

  aggl, aggr, deg0, deg1 = _build_sc_agg()(srcp, dstp, ff, z2, z1, onesh)

  base, p1 = _tc_base(feat, W_r, b_r.reshape(1, 2), U_gcn, b_gcn.reshape(1, D),
                      U_ncn, b_ncn.reshape(1, D))

  out = _tc_comb(aggl, aggr, deg0.reshape(NPAD, 1), deg1.reshape(NPAD, 1),
                 base, p1, W_gcn, W_ncn)
  return out


# speedup vs baseline: 6.6007x; 1.0410x over previous
import functools

import jax
import jax.numpy as jnp
from jax import lax
from jax.experimental import pallas as pl
from jax.experimental.pallas import tpu as pltpu
from jax.experimental.pallas import tpu_sc as plsc

N_NODES = 10000
N_EDGES = 160000
D = 256
DH = 128

NPAD = 10240
ZROWS = NPAD // 16
CHUNK = 112
NCHUNK = 90
EPC = NCHUNK * CHUNK
EPAD = EPC * 16
RB = 3
IR = 6
UNROLL = 6
NOUTER = NCHUNK // UNROLL

BR = 400


def _sc_agg_body(srcp, dstp, ff, z2, z1, onesh,
                 aggl_o, aggr_o, deg0_o, deg1_o,
                 sv0, sv1, sv2, sv3, sv4, sv5,
                 dv0, dv1, dv2, dv3, dv4, dv5,
                 rows0, rows1, rows2, ones_v,
                 agg_sh, deg_sh,
                 is0, is1, is2, is3, is4, is5,
                 gs0, gs1, gs2, ss0, ss1, ss2, dsem):
  cid = lax.axis_index("c")
  sid = lax.axis_index("s")
  r0 = sid * ZROWS
  srcv = (sv0, sv1, sv2, sv3, sv4, sv5)
  dstv = (dv0, dv1, dv2, dv3, dv4, dv5)
  rows = (rows0, rows1, rows2)
  isems = (is0, is1, is2, is3, is4, is5)
  gsems = (gs0, gs1, gs2)
  ssems = (ss0, ss1, ss2)

  pltpu.sync_copy(z2, agg_sh.at[pl.ds(r0, ZROWS)])
  pltpu.sync_copy(z1, deg_sh.at[pl.ds(r0, ZROWS)])
  pltpu.sync_copy(onesh, ones_v)

  def idx_start(i, b):
    pltpu.async_copy(srcp.at[sid, i], srcv[b], isems[b])
    pltpu.async_copy(dstp.at[sid, i], dstv[b], isems[b])

  def idx_wait(b):
    pltpu.make_async_copy(srcp.at[0, 0], srcv[b], isems[b]).wait()
    pltpu.make_async_copy(dstp.at[0, 0], dstv[b], isems[b]).wait()
    for j in range(CHUNK // 16):
      sl = pl.ds(j * 16, 16)
      srcv[b][sl] = srcv[b][sl] * 2 + cid

  def gather_start(i, b, ib):
    pltpu.async_copy(ff.at[srcv[ib]], rows[b], gsems[b])

  def gather_wait(b, ib):
    pltpu.make_async_copy(ff.at[srcv[ib]], rows[b], gsems[b]).wait()

  def scatter_start(b, ib):
    pltpu.async_copy(rows[b], agg_sh.at[dstv[ib]], ssems[b], add=True)

  def scatter_wait(b, ib):
    pltpu.make_async_copy(rows[b], agg_sh.at[dstv[ib]], ssems[b]).wait()

  def deg_start(core, ib):
    @pl.when(cid == core)
    def _():
      pltpu.async_copy(ones_v, deg_sh.at[dstv[ib]], dsem, add=True)

  def deg_wait(core, ib):
    @pl.when(cid == core)
    def _():
      pltpu.make_async_copy(ones_v, deg_sh.at[dstv[ib]], dsem).wait()

  idx_start(0, 0)
  idx_start(1, 1)
  plsc.subcore_barrier()

  def outer(g, carry):
    for u in range(UNROLL):
      i = g * UNROLL + u
      b3 = u % RB
      pb3 = (u + RB - 1) % RB
      pb6 = (u + IR - 1) % IR

      def guard(cond_first, fn):
        if cond_first:
          fn()
        else:
          pl.when(g > 0)(fn)

      idx_wait(u)
      guard(u >= 3, lambda: scatter_wait(b3, (u + IR - 3) % IR))
      gather_start(i, b3, u)
      guard(u >= 1, lambda: gather_wait(pb3, pb6))
      guard(u >= 1, lambda: scatter_start(pb3, pb6))
      dcore = (u + 1) % 2
      guard(u >= 3, lambda: deg_wait(dcore, (u + IR - 3) % IR))
      guard(u >= 1, lambda: deg_start(dcore, pb6))
      if u < 4:
        idx_start(i + 2, (u + 2) % IR)
      else:
        @pl.when(g < NOUTER - 1)
        def _():
          idx_start(i + 2, (u + 2) % IR)
    return carry

  lax.fori_loop(0, NOUTER, outer, 0)

  L = NCHUNK - 1
  gather_wait(L % RB, L % IR)
  scatter_wait((L - 2) % RB, (L - 2) % IR)
  scatter_start(L % RB, L % IR)
  deg_wait(0, (L - 1) % IR)
  deg_wait(1, (L - 2) % IR)
  deg_start(1, L % IR)
  deg_wait(1, L % IR)
  scatter_wait((L - 1) % RB, (L - 1) % IR)
  scatter_wait(L % RB, L % IR)

  plsc.subcore_barrier()

  @pl.when(cid == 0)
  def _():
    pltpu.sync_copy(agg_sh.at[pl.ds(r0, ZROWS)], aggl_o.at[pl.ds(r0, ZROWS)])
    pltpu.sync_copy(deg_sh.at[pl.ds(r0, ZROWS)], deg0_o.at[pl.ds(r0, ZROWS)])

  @pl.when(cid == 1)
  def _():
    pltpu.sync_copy(agg_sh.at[pl.ds(r0, ZROWS)], aggr_o.at[pl.ds(r0, ZROWS)])
    pltpu.sync_copy(deg_sh.at[pl.ds(r0, ZROWS)], deg1_o.at[pl.ds(r0, ZROWS)])


@functools.lru_cache(maxsize=None)
def _build_sc_agg():
  return pl.kernel(
      _sc_agg_body,
      out_type=(
          jax.ShapeDtypeStruct((NPAD, DH), jnp.float32),
          jax.ShapeDtypeStruct((NPAD, DH), jnp.float32),
          jax.ShapeDtypeStruct((NPAD,), jnp.float32),
          jax.ShapeDtypeStruct((NPAD,), jnp.float32),
      ),
      mesh=plsc.VectorSubcoreMesh(core_axis_name="c", subcore_axis_name="s"),
      scratch_types=(
          pltpu.VMEM((CHUNK,), jnp.int32),
          pltpu.VMEM((CHUNK,), jnp.int32),
          pltpu.VMEM((CHUNK,), jnp.int32),
          pltpu.VMEM((CHUNK,), jnp.int32),
          pltpu.VMEM((CHUNK,), jnp.int32),
          pltpu.VMEM((CHUNK,), jnp.int32),
          pltpu.VMEM((CHUNK,), jnp.int32),
          pltpu.VMEM((CHUNK,), jnp.int32),
          pltpu.VMEM((CHUNK,), jnp.int32),
          pltpu.VMEM((CHUNK,), jnp.int32),
          pltpu.VMEM((CHUNK,), jnp.int32),
          pltpu.VMEM((CHUNK,), jnp.int32),
          pltpu.VMEM((CHUNK, DH), jnp.float32),
          pltpu.VMEM((CHUNK, DH), jnp.float32),
          pltpu.VMEM((CHUNK, DH), jnp.float32),
          pltpu.VMEM((CHUNK,), jnp.float32),
          pltpu.VMEM_SHARED((NPAD, DH), jnp.float32),
          pltpu.VMEM_SHARED((NPAD,), jnp.float32),
          pltpu.SemaphoreType.DMA,
          pltpu.SemaphoreType.DMA,
          pltpu.SemaphoreType.DMA,
          pltpu.SemaphoreType.DMA,
          pltpu.SemaphoreType.DMA,
          pltpu.SemaphoreType.DMA,
          pltpu.SemaphoreType.DMA,
          pltpu.SemaphoreType.DMA,
          pltpu.SemaphoreType.DMA,
          pltpu.SemaphoreType.DMA,
          pltpu.SemaphoreType.DMA,
          pltpu.SemaphoreType.DMA,
          pltpu.SemaphoreType.DMA,
      ),
  )


def _tc_base_body(feat_ref, wr_ref, br_ref, ug_ref, bg_ref, un_ref, bn_ref,
                  base_ref, p1_ref):
  f = feat_ref[...]
  dot = functools.partial(jnp.dot, preferred_element_type=jnp.float32,
                          precision=lax.Precision.DEFAULT)
  lg = dot(f, wr_ref[...]) + br_ref[...]
  dlt = lg[:, 1:2] - lg[:, 0:1]
  p1 = 1.0 / (1.0 + jnp.exp(-dlt))
  p0 = 1.0 - p1
  base_ref[...] = (p0 * (dot(f, ug_ref[...]) + bg_ref[...])
                   + p1 * (dot(f, un_ref[...]) + bn_ref[...])).astype(jnp.bfloat16)
  p1_ref[...] = p1


_tc_base = pl.pallas_call(
    _tc_base_body,
    grid=(N_NODES // BR,),
    in_specs=[
        pl.BlockSpec((BR, D), lambda i: (i, 0)),
        pl.BlockSpec((D, 2), lambda i: (0, 0)),
        pl.BlockSpec((1, 2), lambda i: (0, 0)),
        pl.BlockSpec((D, D), lambda i: (0, 0)),
        pl.BlockSpec((1, D), lambda i: (0, 0)),
        pl.BlockSpec((D, D), lambda i: (0, 0)),
        pl.BlockSpec((1, D), lambda i: (0, 0)),
    ],
    out_specs=[
        pl.BlockSpec((BR, D), lambda i: (i, 0)),
        pl.BlockSpec((BR, 1), lambda i: (i, 0)),
    ],
    out_shape=[
        jax.ShapeDtypeStruct((N_NODES, D), jnp.bfloat16),
        jax.ShapeDtypeStruct((N_NODES, 1), jnp.float32),
    ],
)


def _tc_comb_body(aggl_ref, aggr_ref, deg0_ref, deg1_ref, base_ref, p1_ref,
                  wg_ref, wn_ref, out_ref):
  rdeg = 1.0 / jnp.maximum(deg0_ref[...] + deg1_ref[...], 1.0)
  al = aggl_ref[...] * rdeg
  ar = aggr_ref[...] * rdeg
  p1 = p1_ref[...]
  dot = functools.partial(jnp.dot, preferred_element_type=jnp.float32,
                          precision=lax.Precision.DEFAULT)
  gcn = dot(al, wg_ref[0:DH, :]) + dot(ar, wg_ref[DH:D, :])
  ncn = jnp.maximum(dot(al, wn_ref[0:DH, :]) + dot(ar, wn_ref[DH:D, :]), 0.0)
  out_ref[...] = (base_ref[...].astype(jnp.float32)
                  + (1.0 - p1) * gcn + p1 * ncn)


_tc_comb = pl.pallas_call(
    _tc_comb_body,
    grid=(N_NODES // BR,),
    in_specs=[
        pl.BlockSpec((BR, DH), lambda i: (i, 0)),
        pl.BlockSpec((BR, DH), lambda i: (i, 0)),
        pl.BlockSpec((BR, 1), lambda i: (i, 0)),
        pl.BlockSpec((BR, 1), lambda i: (i, 0)),
        pl.BlockSpec((BR, D), lambda i: (i, 0)),
        pl.BlockSpec((BR, 1), lambda i: (i, 0)),
        pl.BlockSpec((D, D), lambda i: (0, 0)),
        pl.BlockSpec((D, D), lambda i: (0, 0)),
    ],
    out_specs=pl.BlockSpec((BR, D), lambda i: (i, 0)),
    out_shape=jax.ShapeDtypeStruct((N_NODES, D), jnp.float32),
)


def kernel(feat, edge_index, W_r, b_r, W_gcn, U_gcn, b_gcn, W_ncn, U_ncn, b_ncn):
  src = edge_index[0]
  dst = edge_index[1]

  pad = EPAD - N_EDGES
  srcp = jnp.concatenate(
      [src, jnp.zeros((pad,), jnp.int32)]).reshape(16, NCHUNK, CHUNK)
  dstp = jnp.concatenate(
      [dst, jnp.full((pad,), N_NODES, jnp.int32)]).reshape(16, NCHUNK, CHUNK)

  ff = feat.reshape(2 * N_NODES, DH)

  z2 = jnp.zeros((ZROWS, DH), jnp.float32)
  z1 = jnp.zeros((ZROWS,), jnp.float32)
  onesh = jnp.ones((CHUNK,), jnp.float32)

  aggl, aggr, deg0, deg1 = _build_sc_agg()(srcp, dstp, ff, z2, z1, onesh)

  base, p1 = _tc_base(feat, W_r, b_r.reshape(1, 2), U_gcn, b_gcn.reshape(1, D),
                      U_ncn, b_ncn.reshape(1, D))

  out = _tc_comb(aggl, aggr, deg0.reshape(NPAD, 1), deg1.reshape(NPAD, 1),
                 base, p1, W_gcn, W_ncn)
  return out
